# Initial kernel scaffold; baseline (speedup 1.0000x reference)
import jax
import jax.numpy as jnp
from jax.experimental import pallas as pl


def kernel(s, v, edge_index, edge_dist, edge_vector, W1, b1, W2, b2, Wf, bf):
    raise NotImplementedError("write your pallas kernel here")


# speedup vs baseline: 4.6317x; 4.6317x over previous
import functools

import jax
import jax.numpy as jnp
from jax import lax
from jax.experimental import pallas as pl
from jax.experimental.pallas import tpu as pltpu
from jax.experimental.pallas import tpu_sc as plsc

L = 16
NTILES = 16
NB = 32
ZR = 40


def _mlp_pallas(s, W1t, b1, W2t, b2):
    N, H = s.shape
    TH = W2t.shape[1]
    R = 400
    assert N % R == 0

    def body(s_ref, w1_ref, b1_ref, w2_ref, b2_ref, h0_ref, h12_ref):
        t = jnp.dot(s_ref[...], w1_ref[...], preferred_element_type=jnp.float32)
        t = jnp.maximum(t + b1_ref[...], 0.0)
        hh = jnp.dot(t, w2_ref[...], preferred_element_type=jnp.float32)
        hh = hh + b2_ref[...]
        h0_ref[...] = hh[:, :H]
        h12_ref[...] = hh[:, H:]

    return pl.pallas_call(
        body,
        grid=(N // R,),
        in_specs=[
            pl.BlockSpec((R, H), lambda i: (i, 0)),
            pl.BlockSpec((H, H), lambda i: (0, 0)),
            pl.BlockSpec((1, H), lambda i: (0, 0)),
            pl.BlockSpec((H, TH), lambda i: (0, 0)),
            pl.BlockSpec((1, TH), lambda i: (0, 0)),
        ],
        out_specs=[
            pl.BlockSpec((R, H), lambda i: (i, 0)),
            pl.BlockSpec((R, TH - H), lambda i: (i, 0)),
        ],
        out_shape=[
            jax.ShapeDtypeStruct((N, H), jnp.float32),
            jax.ShapeDtypeStruct((N, TH - H), jnp.float32),
        ],
    )(s, W1t, b1, W2t, b2)


def _filter_pallas(edge_dist, Wft, bf):
    E, G = edge_dist.shape
    TH = Wft.shape[1]
    H = TH // 3
    R = 2000
    assert E % R == 0

    def body(d_ref, wf_ref, bf_ref, w0_ref, w12_ref):
        ww = jnp.dot(d_ref[...], wf_ref[...], preferred_element_type=jnp.float32)
        ww = ww + bf_ref[...]
        w0_ref[...] = ww[:, :H]
        w12_ref[...] = ww[:, H:]

    return pl.pallas_call(
        body,
        grid=(E // R,),
        in_specs=[
            pl.BlockSpec((R, G), lambda i: (i, 0)),
            pl.BlockSpec((G, TH), lambda i: (0, 0)),
            pl.BlockSpec((1, TH), lambda i: (0, 0)),
        ],
        out_specs=[
            pl.BlockSpec((R, H), lambda i: (i, 0)),
            pl.BlockSpec((R, TH - H), lambda i: (i, 0)),
        ],
        out_shape=[
            jax.ShapeDtypeStruct((E, H), jnp.float32),
            jax.ShapeDtypeStruct((E, TH - H), jnp.float32),
        ],
    )(edge_dist, Wft, bf)


@functools.cache
def _make_sc_kernel(N, E, H):
    assert N % ZR == 0
    assert E % (NTILES * NB) == 0
    nch = N // ZR
    nch_pt = -(-nch // NTILES)
    ept = E // NTILES
    nbatch = ept // NB
    nslc = H // L
    mesh = plsc.VectorSubcoreMesh(core_axis_name="c", subcore_axis_name="s",
                                  num_cores=2, num_subcores=NTILES)

    @functools.partial(
        pl.kernel,
        out_type=[jax.ShapeDtypeStruct((N, H), jnp.float32)] * 4,
        mesh=mesh,
        scratch_types=[
            pltpu.VMEM_SHARED((N, H), jnp.float32),
            pltpu.VMEM((NB,), jnp.int32),
            pltpu.VMEM((NB,), jnp.int32),
            pltpu.VMEM((NB, 2 * H), jnp.float32),
            pltpu.VMEM((NB, 2 * H), jnp.float32),
            pltpu.VMEM((NB, H), jnp.float32),
            pltpu.VMEM((NB, H), jnp.float32),
            pltpu.VMEM((NB,), jnp.float32),
            pltpu.VMEM((NB, H), jnp.float32),
            pltpu.VMEM((ZR, H), jnp.float32),
            pltpu.SemaphoreType.DMA,
        ],
    )
    def sc_kernel(h0, h12, w0, w12, v0, v1, v2, ev0, ev1, ev2, src, dst,
                  s_out, u0_out, u1_out, u2_out,
                  acc, srcb, dstb, hbuf, wbuf, vkbuf, w0buf, evb, contrib,
                  zbuf, sem):
        cid = lax.axis_index("c")
        sid = lax.axis_index("s")
        zero16 = jnp.zeros((L,), jnp.float32)

        def fill_zbuf(i, c):
            for j in range(nslc):
                zbuf[i, pl.ds(L * j, L)] = zero16
            return c

        def zero_acc():
            lax.fori_loop(0, ZR, fill_zbuf, 0)

            def z(i, c):
                cidx = sid + NTILES * i

                @pl.when(cidx < nch)
                def _():
                    pltpu.sync_copy(zbuf, acc.at[pl.ds(cidx * ZR, ZR)])
                return c
            lax.fori_loop(0, nch_pt, z, 0)

        def writeout(out_hbm):
            def wlp(i, c):
                cidx = sid + NTILES * i

                @pl.when(cidx < nch)
                def _():
                    r0 = cidx * ZR
                    pltpu.sync_copy(acc.at[pl.ds(r0, ZR)], zbuf)
                    pltpu.sync_copy(zbuf, out_hbm.at[pl.ds(r0, ZR)])
                return c
            lax.fori_loop(0, nch_pt, wlp, 0)

        def accum_s():
            def batch(b, c):
                base = sid * ept + b * NB
                pltpu.sync_copy(dst.at[pl.ds(base, NB)], dstb)
                pltpu.sync_copy(src.at[pl.ds(base, NB)], srcb)
                pltpu.sync_copy(w0.at[pl.ds(base, NB)], w0buf)
                pltpu.async_copy(h0.at[srcb], vkbuf, sem).wait()

                def edge(e, c2):
                    for j in range(nslc):
                        sl = pl.ds(L * j, L)
                        contrib[e, sl] = vkbuf[e, sl] * w0buf[e, sl]
                    return c2

                lax.fori_loop(0, NB, edge, 0)
                pltpu.sync_copy(contrib, acc.at[dstb], add=True)
                return c
            lax.fori_loop(0, nbatch, batch, 0)

        def accum_v(vk, evk):
            def batch(b, c):
                base = sid * ept + b * NB
                pltpu.sync_copy(dst.at[pl.ds(base, NB)], dstb)
                pltpu.sync_copy(src.at[pl.ds(base, NB)], srcb)
                pltpu.sync_copy(w12.at[pl.ds(base, NB)], wbuf)
                pltpu.sync_copy(evk.at[pl.ds(base, NB)], evb)
                pltpu.async_copy(h12.at[srcb], hbuf, sem).wait()
                pltpu.async_copy(vk.at[srcb], vkbuf, sem).wait()

                def edge_group(g, c2):
                    ev16 = evb[pl.ds(L * g, L)]
                    for t in range(L):
                        e = L * g + t
                        ev = ev16[t]
                        for j in range(nslc):
                            sl = pl.ds(L * j, L)
                            sl2 = pl.ds(H + L * j, L)
                            g1 = hbuf[e, sl] * wbuf[e, sl]
                            g2 = hbuf[e, sl2] * wbuf[e, sl2]
                            contrib[e, sl] = vkbuf[e, sl] * g1 + ev * g2
                    return c2

                lax.fori_loop(0, NB // L, edge_group, 0)
                pltpu.sync_copy(contrib, acc.at[dstb], add=True)
                return c
            lax.fori_loop(0, nbatch, batch, 0)

        def chunk(accum_fn, out_hbm):
            zero_acc()
            plsc.subcore_barrier()
            accum_fn()
            plsc.subcore_barrier()
            writeout(out_hbm)
            plsc.subcore_barrier()

        @pl.when(cid == 0)
        def _():
            chunk(accum_s, s_out)
            chunk(lambda: accum_v(v0, ev0), u0_out)

        @pl.when(cid == 1)
        def _():
            chunk(lambda: accum_v(v1, ev1), u1_out)
            chunk(lambda: accum_v(v2, ev2), u2_out)

    return sc_kernel


def kernel(s, v, edge_index, edge_dist, edge_vector, W1, b1, W2, b2, Wf, bf):
    N, H = s.shape
    E = edge_index.shape[1]
    src = edge_index[0]
    dst = edge_index[1]

    h0, h12 = _mlp_pallas(s, W1.T, b1.reshape(1, H), W2.T, b2.reshape(1, 3 * H))
    w0, w12 = _filter_pallas(edge_dist, Wf.T, bf.reshape(1, 3 * H))

    v0 = v[:, 0, :]
    v1 = v[:, 1, :]
    v2 = v[:, 2, :]
    ev0 = edge_vector[:, 0]
    ev1 = edge_vector[:, 1]
    ev2 = edge_vector[:, 2]

    s_out, u0, u1, u2 = _make_sc_kernel(N, E, H)(
        h0, h12, w0, w12, v0, v1, v2, ev0, ev1, ev2, src, dst)
    v_out = jnp.stack([u0, u1, u2], axis=1)
    return (s_out, v_out)


# speedup vs baseline: 7.3406x; 1.5849x over previous
import functools

import jax
import jax.numpy as jnp
from jax import lax
from jax.experimental import pallas as pl
from jax.experimental.pallas import tpu as pltpu
from jax.experimental.pallas import tpu_sc as plsc

L = 16
NTILES = 16
NB = 32
ZR = 16


def _mlp_pallas(s, W1t, b1, W2t, b2):
    N, H = s.shape
    TH = W2t.shape[1]
    R = 400
    assert N % R == 0

    def body(s_ref, w1_ref, b1_ref, w2_ref, b2_ref, h0_ref, h12_ref):
        t = jnp.dot(s_ref[...], w1_ref[...], preferred_element_type=jnp.float32)
        t = jnp.maximum(t + b1_ref[...], 0.0)
        hh = jnp.dot(t, w2_ref[...], preferred_element_type=jnp.float32)
        hh = hh + b2_ref[...]
        h0_ref[...] = hh[:, :H]
        h12_ref[...] = hh[:, H:]

    return pl.pallas_call(
        body,
        grid=(N // R,),
        in_specs=[
            pl.BlockSpec((R, H), lambda i: (i, 0)),
            pl.BlockSpec((H, H), lambda i: (0, 0)),
            pl.BlockSpec((1, H), lambda i: (0, 0)),
            pl.BlockSpec((H, TH), lambda i: (0, 0)),
            pl.BlockSpec((1, TH), lambda i: (0, 0)),
        ],
        out_specs=[
            pl.BlockSpec((R, H), lambda i: (i, 0)),
            pl.BlockSpec((R, TH - H), lambda i: (i, 0)),
        ],
        out_shape=[
            jax.ShapeDtypeStruct((N, H), jnp.float32),
            jax.ShapeDtypeStruct((N, TH - H), jnp.float32),
        ],
    )(s, W1t, b1, W2t, b2)


def _filter_pallas(edge_dist, Wft, bf):
    E, G = edge_dist.shape
    TH = Wft.shape[1]
    H = TH // 3
    R = 2000
    assert E % R == 0

    def body(d_ref, wf_ref, bf_ref, w0_ref, w12_ref):
        ww = jnp.dot(d_ref[...], wf_ref[...], preferred_element_type=jnp.float32)
        ww = ww + bf_ref[...]
        w0_ref[...] = ww[:, :H]
        w12_ref[...] = ww[:, H:]

    return pl.pallas_call(
        body,
        grid=(E // R,),
        in_specs=[
            pl.BlockSpec((R, G), lambda i: (i, 0)),
            pl.BlockSpec((G, TH), lambda i: (0, 0)),
            pl.BlockSpec((1, TH), lambda i: (0, 0)),
        ],
        out_specs=[
            pl.BlockSpec((R, H), lambda i: (i, 0)),
            pl.BlockSpec((R, TH - H), lambda i: (i, 0)),
        ],
        out_shape=[
            jax.ShapeDtypeStruct((E, H), jnp.float32),
            jax.ShapeDtypeStruct((E, TH - H), jnp.float32),
        ],
    )(edge_dist, Wft, bf)


@functools.cache
def _make_sc_kernel(N, E, H):
    assert N % ZR == 0
    assert E % (NTILES * NB) == 0
    nch = N // ZR
    nch_pt = -(-nch // NTILES)
    ept = E // NTILES
    nbatch = ept // NB
    assert nbatch % 2 == 1
    nslc = H // L
    mesh = plsc.VectorSubcoreMesh(core_axis_name="c", subcore_axis_name="s",
                                  num_cores=2, num_subcores=NTILES)

    @functools.partial(
        pl.kernel,
        out_type=[jax.ShapeDtypeStruct((N, H), jnp.float32)] * 4,
        mesh=mesh,
        scratch_types=[
            pltpu.VMEM_SHARED((N, H), jnp.float32),
            pltpu.VMEM((NB,), jnp.int32),
            pltpu.VMEM((NB,), jnp.int32),
            pltpu.VMEM((NB,), jnp.float32),
            pltpu.VMEM((NB, 2 * H), jnp.float32),
            pltpu.VMEM((NB, 2 * H), jnp.float32),
            pltpu.VMEM((NB, H), jnp.float32),
            pltpu.VMEM((NB,), jnp.int32),
            pltpu.VMEM((NB,), jnp.int32),
            pltpu.VMEM((NB,), jnp.float32),
            pltpu.VMEM((NB, 2 * H), jnp.float32),
            pltpu.VMEM((NB, 2 * H), jnp.float32),
            pltpu.VMEM((NB, H), jnp.float32),
            pltpu.VMEM((NB, H), jnp.float32),
            pltpu.VMEM((ZR, H), jnp.float32),
            pltpu.SemaphoreType.DMA,
            pltpu.SemaphoreType.DMA,
            pltpu.SemaphoreType.DMA,
            pltpu.SemaphoreType.DMA,
        ],
    )
    def sc_kernel(h0, h12, w0, w12, v0, v1, v2, ev0, ev1, ev2, src, dst,
                  s_out, u0_out, u1_out, u2_out,
                  acc,
                  srcb0, dstb0, evb0, hbuf0, wbuf0, vkbuf0,
                  srcb1, dstb1, evb1, hbuf1, wbuf1, vkbuf1,
                  w0buf, zbuf, semL0, semG0, semL1, semG1):
        sets = ((srcb0, dstb0, evb0, hbuf0, wbuf0, vkbuf0, semL0, semG0),
                (srcb1, dstb1, evb1, hbuf1, wbuf1, vkbuf1, semL1, semG1))
        cid = lax.axis_index("c")
        sid = lax.axis_index("s")
        zero16 = jnp.zeros((L,), jnp.float32)

        def fill_zbuf(i, c):
            for j in range(nslc):
                zbuf[i, pl.ds(L * j, L)] = zero16
            return c

        def zero_acc():
            lax.fori_loop(0, ZR, fill_zbuf, 0)

            def z(i, c):
                cidx = sid + NTILES * i

                @pl.when(cidx < nch)
                def _():
                    pltpu.sync_copy(zbuf, acc.at[pl.ds(cidx * ZR, ZR)])
                return c
            lax.fori_loop(0, nch_pt, z, 0)

        def writeout(out_hbm):
            def wlp(i, c):
                cidx = sid + NTILES * i

                @pl.when(cidx < nch)
                def _():
                    r0 = cidx * ZR
                    pltpu.sync_copy(acc.at[pl.ds(r0, ZR)], zbuf)
                    pltpu.sync_copy(zbuf, out_hbm.at[pl.ds(r0, ZR)])
                return c
            lax.fori_loop(0, nch_pt, wlp, 0)

        def accum_s():
            srcb, dstb, _, hbuf, _, vkbuf, semL, semG = sets[0]

            def batch(b, c):
                base = sid * ept + b * NB
                pltpu.sync_copy(dst.at[pl.ds(base, NB)], dstb)
                pltpu.sync_copy(src.at[pl.ds(base, NB)], srcb)
                pltpu.sync_copy(w0.at[pl.ds(base, NB)], w0buf)
                pltpu.async_copy(h0.at[srcb], vkbuf, semG).wait()

                def edge(e, c2):
                    for j in range(nslc):
                        sl = pl.ds(L * j, L)
                        vkbuf[e, sl] = vkbuf[e, sl] * w0buf[e, sl]
                    return c2

                lax.fori_loop(0, NB, edge, 0)
                pltpu.sync_copy(vkbuf, acc.at[dstb], add=True)
                return c
            lax.fori_loop(0, nbatch, batch, 0)

        def accum_v(vk, evk):
            def issue_linear(b, st):
                srcb, dstb, evb, _, wbuf, _, semL, _ = st
                base = sid * ept + b * NB
                pltpu.async_copy(dst.at[pl.ds(base, NB)], dstb, semL)
                pltpu.async_copy(src.at[pl.ds(base, NB)], srcb, semL)
                pltpu.async_copy(w12.at[pl.ds(base, NB)], wbuf, semL)
                pltpu.async_copy(evk.at[pl.ds(base, NB)], evb, semL)

            def wait_linear(b, st):
                srcb, dstb, evb, _, wbuf, _, semL, _ = st
                base = sid * ept + b * NB
                pltpu.make_async_copy(dst.at[pl.ds(base, NB)], dstb, semL).wait()
                pltpu.make_async_copy(src.at[pl.ds(base, NB)], srcb, semL).wait()
                pltpu.make_async_copy(w12.at[pl.ds(base, NB)], wbuf, semL).wait()
                pltpu.make_async_copy(evk.at[pl.ds(base, NB)], evb, semL).wait()

            def issue_gather(st):
                srcb, _, _, hbuf, _, vkbuf, _, semG = st
                pltpu.async_copy(h12.at[srcb], hbuf, semG)
                pltpu.async_copy(vk.at[srcb], vkbuf, semG)

            def wait_gather(st):
                srcb, _, _, hbuf, _, vkbuf, _, semG = st
                pltpu.make_async_copy(h12.at[srcb], hbuf, semG).wait()
                pltpu.make_async_copy(vk.at[srcb], vkbuf, semG).wait()

            def compute_scatter(st):
                _, dstb, evb, hbuf, wbuf, vkbuf, _, _ = st

                def edge_group(g, c2):
                    ev16 = evb[pl.ds(L * g, L)]
                    evs = [ev16[t] for t in range(L)]

                    def jbody(j, c3):
                        sl = pl.ds(L * j, L)
                        sl2 = pl.ds(H + L * j, L)
                        for t in range(L):
                            e = L * g + t
                            g1 = hbuf[e, sl] * wbuf[e, sl]
                            g2 = hbuf[e, sl2] * wbuf[e, sl2]
                            vkbuf[e, sl] = vkbuf[e, sl] * g1 + evs[t] * g2
                        return c3

                    lax.fori_loop(0, nslc, jbody, 0)
                    return c2

                lax.fori_loop(0, NB // L, edge_group, 0)
                pltpu.sync_copy(vkbuf, acc.at[dstb], add=True)

            issue_linear(0, sets[0])
            wait_linear(0, sets[0])
            issue_gather(sets[0])

            def pair(g2, c):
                b0 = 2 * g2
                issue_linear(b0 + 1, sets[1])
                wait_gather(sets[0])
                compute_scatter(sets[0])
                wait_linear(b0 + 1, sets[1])
                issue_gather(sets[1])
                issue_linear(b0 + 2, sets[0])
                wait_gather(sets[1])
                compute_scatter(sets[1])
                wait_linear(b0 + 2, sets[0])
                issue_gather(sets[0])
                return c

            lax.fori_loop(0, (nbatch - 1) // 2, pair, 0)
            wait_gather(sets[0])
            compute_scatter(sets[0])

        def chunk(accum_fn, out_hbm):
            zero_acc()
            plsc.subcore_barrier()
            accum_fn()
            plsc.subcore_barrier()
            writeout(out_hbm)
            plsc.subcore_barrier()

        @pl.when(cid == 0)
        def _():
            chunk(accum_s, s_out)
            chunk(lambda: accum_v(v0, ev0), u0_out)

        @pl.when(cid == 1)
        def _():
            chunk(lambda: accum_v(v1, ev1), u1_out)
            chunk(lambda: accum_v(v2, ev2), u2_out)

    return sc_kernel


def kernel(s, v, edge_index, edge_dist, edge_vector, W1, b1, W2, b2, Wf, bf):
    N, H = s.shape
    E = edge_index.shape[1]
    src = edge_index[0]
    dst = edge_index[1]

    h0, h12 = _mlp_pallas(s, W1.T, b1.reshape(1, H), W2.T, b2.reshape(1, 3 * H))
    w0, w12 = _filter_pallas(edge_dist, Wf.T, bf.reshape(1, 3 * H))

    v0 = v[:, 0, :]
    v1 = v[:, 1, :]
    v2 = v[:, 2, :]
    ev0 = edge_vector[:, 0]
    ev1 = edge_vector[:, 1]
    ev2 = edge_vector[:, 2]

    s_out, u0, u1, u2 = _make_sc_kernel(N, E, H)(
        h0, h12, w0, w12, v0, v1, v2, ev0, ev1, ev2, src, dst)
    v_out = jnp.stack([u0, u1, u2], axis=1)
    return (s_out, v_out)


# speedup vs baseline: 9.3407x; 1.2725x over previous
import functools

import jax
import jax.numpy as jnp
from jax import lax
from jax.experimental import pallas as pl
from jax.experimental.pallas import tpu as pltpu
from jax.experimental.pallas import tpu_sc as plsc

L = 16
NTILES = 16
NB = 32
ZR = 16


def _mlp_pallas(s, W1t, b1, W2t, b2):
    N, H = s.shape
    TH = W2t.shape[1]
    R = 400
    assert N % R == 0

    def body(s_ref, w1_ref, b1_ref, w2_ref, b2_ref, h0_ref, h12_ref):
        t = jnp.dot(s_ref[...], w1_ref[...], preferred_element_type=jnp.float32)
        t = jnp.maximum(t + b1_ref[...], 0.0)
        hh = jnp.dot(t, w2_ref[...], preferred_element_type=jnp.float32)
        hh = hh + b2_ref[...]
        h0_ref[...] = hh[:, :H]
        h12_ref[...] = hh[:, H:]

    return pl.pallas_call(
        body,
        grid=(N // R,),
        in_specs=[
            pl.BlockSpec((R, H), lambda i: (i, 0)),
            pl.BlockSpec((H, H), lambda i: (0, 0)),
            pl.BlockSpec((1, H), lambda i: (0, 0)),
            pl.BlockSpec((H, TH), lambda i: (0, 0)),
            pl.BlockSpec((1, TH), lambda i: (0, 0)),
        ],
        out_specs=[
            pl.BlockSpec((R, H), lambda i: (i, 0)),
            pl.BlockSpec((R, TH - H), lambda i: (i, 0)),
        ],
        out_shape=[
            jax.ShapeDtypeStruct((N, H), jnp.float32),
            jax.ShapeDtypeStruct((N, TH - H), jnp.float32),
        ],
    )(s, W1t, b1, W2t, b2)


def _filter_pallas(edge_dist, Wft, bf):
    E, G = edge_dist.shape
    TH = Wft.shape[1]
    H = TH // 3
    R = 2000
    assert E % R == 0

    def body(d_ref, wf_ref, bf_ref, w0_ref, w12_ref):
        ww = jnp.dot(d_ref[...], wf_ref[...], preferred_element_type=jnp.float32)
        ww = ww + bf_ref[...]
        w0_ref[...] = ww[:, :H]
        w12_ref[...] = ww[:, H:]

    return pl.pallas_call(
        body,
        grid=(E // R,),
        in_specs=[
            pl.BlockSpec((R, G), lambda i: (i, 0)),
            pl.BlockSpec((G, TH), lambda i: (0, 0)),
            pl.BlockSpec((1, TH), lambda i: (0, 0)),
        ],
        out_specs=[
            pl.BlockSpec((R, H), lambda i: (i, 0)),
            pl.BlockSpec((R, TH - H), lambda i: (i, 0)),
        ],
        out_shape=[
            jax.ShapeDtypeStruct((E, H), jnp.float32),
            jax.ShapeDtypeStruct((E, TH - H), jnp.float32),
        ],
    )(edge_dist, Wft, bf)


@functools.cache
def _make_sc_kernel(N, E, H):
    assert N % ZR == 0
    assert E % (NTILES * NB) == 0
    nch = N // ZR
    nch_pt = -(-nch // NTILES)
    ept = E // NTILES
    nbatch = ept // NB
    NBS = L
    nbatch_s = ept // NBS
    nslc = H // L
    mesh = plsc.VectorSubcoreMesh(core_axis_name="c", subcore_axis_name="s",
                                  num_cores=2, num_subcores=NTILES)

    @functools.partial(
        pl.kernel,
        out_type=[jax.ShapeDtypeStruct((N, H), jnp.float32)] * 4,
        mesh=mesh,
        scratch_types=[
            pltpu.VMEM_SHARED((N, H), jnp.float32),
            pltpu.VMEM((NB,), jnp.int32),
            pltpu.VMEM((NB,), jnp.int32),
            pltpu.VMEM((NB,), jnp.float32),
            pltpu.VMEM((NB, 2 * H), jnp.float32),
            pltpu.VMEM((NB, 2 * H), jnp.float32),
            pltpu.VMEM((NB, H), jnp.float32),
            pltpu.VMEM((NB,), jnp.int32),
            pltpu.VMEM((NB,), jnp.int32),
            pltpu.VMEM((NB,), jnp.float32),
            pltpu.VMEM((NB, 2 * H), jnp.float32),
            pltpu.VMEM((NB, 2 * H), jnp.float32),
            pltpu.VMEM((NB, H), jnp.float32),
            pltpu.VMEM((L,), jnp.int32),
            pltpu.VMEM((L,), jnp.int32),
            pltpu.VMEM((L,), jnp.int32),
            pltpu.VMEM((L,), jnp.int32),
            pltpu.VMEM((ZR, H), jnp.float32),
            pltpu.SemaphoreType.DMA,
            pltpu.SemaphoreType.DMA,
            pltpu.SemaphoreType.DMA,
            pltpu.SemaphoreType.DMA,
        ],
    )
    def sc_kernel(h0, h12, w0, w12, v0, v1, v2, ev0, ev1, ev2, src, dst,
                  s_out, u0_out, u1_out, u2_out,
                  acc,
                  srcb0, dstb0, evb0, hbuf0, wbuf0, vkbuf0,
                  srcb1, dstb1, evb1, hbuf1, wbuf1, vkbuf1,
                  srcbs0, dstbs0, srcbs1, dstbs1,
                  zbuf, semS0, semB0, semS1, semB1):
        sets = ((srcb0, dstb0, evb0, hbuf0, wbuf0, vkbuf0, semS0, semB0),
                (srcb1, dstb1, evb1, hbuf1, wbuf1, vkbuf1, semS1, semB1))
        cid = lax.axis_index("c")
        sid = lax.axis_index("s")
        zero16 = jnp.zeros((L,), jnp.float32)

        def fill_zbuf(i, c):
            for j in range(nslc):
                zbuf[i, pl.ds(L * j, L)] = zero16
            return c

        def zero_acc():
            lax.fori_loop(0, ZR, fill_zbuf, 0)

            def z(i, c):
                cidx = sid + NTILES * i

                @pl.when(cidx < nch)
                def _():
                    pltpu.sync_copy(zbuf, acc.at[pl.ds(cidx * ZR, ZR)])
                return c
            lax.fori_loop(0, nch_pt, z, 0)

        def writeout(out_hbm):
            def wlp(i, c):
                cidx = sid + NTILES * i

                @pl.when(cidx < nch)
                def _():
                    r0 = cidx * ZR
                    pltpu.sync_copy(acc.at[pl.ds(r0, ZR)], zbuf)
                    pltpu.sync_copy(zbuf, out_hbm.at[pl.ds(r0, ZR)])
                return c
            lax.fori_loop(0, nch_pt, wlp, 0)

        def pipelined(nb_total, issue_small, wait_small, issue_big, wait_big,
                      do_batch):
            issue_small(0, 0)
            wait_small(0, 0)
            issue_big(0, 0)
            if nb_total > 1:
                issue_small(1, 1)

            def phase(b, slot):
                @pl.when(b + 1 < nb_total)
                def _():
                    wait_small(b + 1, 1 - slot)
                    issue_big(b + 1, 1 - slot)
                wait_big(b, slot)
                do_batch(b, slot)

                @pl.when(b + 2 < nb_total)
                def _():
                    issue_small(b + 2, slot)

            def pair(i, c):
                b0 = 2 * i
                phase(b0, 0)

                @pl.when(b0 + 1 < nb_total)
                def _():
                    phase(b0 + 1, 1)
                return c

            lax.fori_loop(0, (nb_total + 1) // 2, pair, 0)

        def accum_s():
            smalls = ((srcbs0, dstbs0, semS0), (srcbs1, dstbs1, semS1))
            bigs = ((vkbuf0, semB0), (vkbuf1, semB1))

            def issue_small(b, k):
                srcbs, dstbs, semS = smalls[k]
                base = sid * ept + b * NBS
                pltpu.async_copy(dst.at[pl.ds(base, NBS)], dstbs, semS)
                pltpu.async_copy(src.at[pl.ds(base, NBS)], srcbs, semS)

            def wait_small(b, k):
                srcbs, dstbs, semS = smalls[k]
                base = sid * ept + b * NBS
                pltpu.make_async_copy(dst.at[pl.ds(base, NBS)], dstbs, semS).wait()
                pltpu.make_async_copy(src.at[pl.ds(base, NBS)], srcbs, semS).wait()

            def issue_big(b, k):
                srcbs = smalls[k][0]
                vkbuf, semB = bigs[k]
                base = sid * ept + b * NBS
                pltpu.async_copy(h0.at[srcbs], vkbuf.at[pl.ds(0, NBS)], semB)
                pltpu.async_copy(w0.at[pl.ds(base, NBS)],
                                 vkbuf.at[pl.ds(NBS, NBS)], semB)

            def wait_big(b, k):
                srcbs = smalls[k][0]
                vkbuf, semB = bigs[k]
                base = sid * ept + b * NBS
                pltpu.make_async_copy(h0.at[srcbs],
                                      vkbuf.at[pl.ds(0, NBS)], semB).wait()
                pltpu.make_async_copy(w0.at[pl.ds(base, NBS)],
                                      vkbuf.at[pl.ds(NBS, NBS)], semB).wait()

            def do_batch(b, k):
                dstbs = smalls[k][1]
                vkbuf = bigs[k][0]

                def edge(e, c2):
                    for j in range(nslc):
                        sl = pl.ds(L * j, L)
                        vkbuf[e, sl] = vkbuf[e, sl] * vkbuf[NBS + e, sl]
                    return c2

                lax.fori_loop(0, NBS, edge, 0)
                pltpu.sync_copy(vkbuf.at[pl.ds(0, NBS)], acc.at[dstbs], add=True)

            pipelined(nbatch_s, issue_small, wait_small, issue_big, wait_big,
                      do_batch)

        def accum_v(vk, evk):
            def issue_small(b, k):
                srcb, dstb, evb = sets[k][0], sets[k][1], sets[k][2]
                semS = sets[k][6]
                base = sid * ept + b * NB
                pltpu.async_copy(dst.at[pl.ds(base, NB)], dstb, semS)
                pltpu.async_copy(src.at[pl.ds(base, NB)], srcb, semS)
                pltpu.async_copy(evk.at[pl.ds(base, NB)], evb, semS)

            def wait_small(b, k):
                srcb, dstb, evb = sets[k][0], sets[k][1], sets[k][2]
                semS = sets[k][6]
                base = sid * ept + b * NB
                pltpu.make_async_copy(dst.at[pl.ds(base, NB)], dstb, semS).wait()
                pltpu.make_async_copy(src.at[pl.ds(base, NB)], srcb, semS).wait()
                pltpu.make_async_copy(evk.at[pl.ds(base, NB)], evb, semS).wait()

            def issue_big(b, k):
                srcb, hbuf, wbuf, vkbuf = (sets[k][0], sets[k][3], sets[k][4],
                                           sets[k][5])
                semB = sets[k][7]
                base = sid * ept + b * NB
                pltpu.async_copy(h12.at[srcb], hbuf, semB)
                pltpu.async_copy(vk.at[srcb], vkbuf, semB)
                pltpu.async_copy(w12.at[pl.ds(base, NB)], wbuf, semB)

            def wait_big(b, k):
                srcb, hbuf, wbuf, vkbuf = (sets[k][0], sets[k][3], sets[k][4],
                                           sets[k][5])
                semB = sets[k][7]
                base = sid * ept + b * NB
                pltpu.make_async_copy(h12.at[srcb], hbuf, semB).wait()
                pltpu.make_async_copy(vk.at[srcb], vkbuf, semB).wait()
                pltpu.make_async_copy(w12.at[pl.ds(base, NB)], wbuf, semB).wait()

            def do_batch(b, k):
                dstb, evb, hbuf, wbuf, vkbuf = (sets[k][1], sets[k][2],
                                                sets[k][3], sets[k][4],
                                                sets[k][5])

                def edge_group(g, c2):
                    ev16 = evb[pl.ds(L * g, L)]
                    evs = [ev16[t] for t in range(L)]

                    def jbody(j, c3):
                        sl = pl.ds(L * j, L)
                        sl2 = pl.ds(H + L * j, L)
                        for t in range(L):
                            e = L * g + t
                            g1 = hbuf[e, sl] * wbuf[e, sl]
                            g2 = hbuf[e, sl2] * wbuf[e, sl2]
                            vkbuf[e, sl] = vkbuf[e, sl] * g1 + evs[t] * g2
                        return c3

                    lax.fori_loop(0, nslc, jbody, 0)
                    return c2

                lax.fori_loop(0, NB // L, edge_group, 0)
                pltpu.sync_copy(vkbuf, acc.at[dstb], add=True)

            pipelined(nbatch, issue_small, wait_small, issue_big, wait_big,
                      do_batch)

        def chunk(accum_fn, out_hbm):
            zero_acc()
            plsc.subcore_barrier()
            accum_fn()
            plsc.subcore_barrier()
            writeout(out_hbm)
            plsc.subcore_barrier()

        @pl.when(cid == 0)
        def _():
            chunk(accum_s, s_out)
            chunk(lambda: accum_v(v0, ev0), u0_out)

        @pl.when(cid == 1)
        def _():
            chunk(lambda: accum_v(v1, ev1), u1_out)
            chunk(lambda: accum_v(v2, ev2), u2_out)

    return sc_kernel


def kernel(s, v, edge_index, edge_dist, edge_vector, W1, b1, W2, b2, Wf, bf):
    N, H = s.shape
    E = edge_index.shape[1]
    src = edge_index[0]
    dst = edge_index[1]

    h0, h12 = _mlp_pallas(s, W1.T, b1.reshape(1, H), W2.T, b2.reshape(1, 3 * H))
    w0, w12 = _filter_pallas(edge_dist, Wf.T, bf.reshape(1, 3 * H))

    v0 = v[:, 0, :]
    v1 = v[:, 1, :]
    v2 = v[:, 2, :]
    ev0 = edge_vector[:, 0]
    ev1 = edge_vector[:, 1]
    ev2 = edge_vector[:, 2]

    s_out, u0, u1, u2 = _make_sc_kernel(N, E, H)(
        h0, h12, w0, w12, v0, v1, v2, ev0, ev1, ev2, src, dst)
    v_out = jnp.stack([u0, u1, u2], axis=1)
    return (s_out, v_out)


# speedup vs baseline: 12.2444x; 1.3109x over previous
import functools

import jax
import jax.numpy as jnp
from jax import lax
from jax.experimental import pallas as pl
from jax.experimental.pallas import tpu as pltpu
from jax.experimental.pallas import tpu_sc as plsc

L = 16
NTILES = 16
NB = 16
NR = 4
ZR = 16


def _mlp_pallas(s, W1t, b1, W2t, b2):
    N, H = s.shape
    TH = W2t.shape[1]
    R = 400
    assert N % R == 0

    def body(s_ref, w1_ref, b1_ref, w2_ref, b2_ref, h0_ref, h12_ref):
        t = jnp.dot(s_ref[...], w1_ref[...], preferred_element_type=jnp.float32)
        t = jnp.maximum(t + b1_ref[...], 0.0)
        hh = jnp.dot(t, w2_ref[...], preferred_element_type=jnp.float32)
        hh = hh + b2_ref[...]
        h0_ref[...] = hh[:, :H]
        h12_ref[...] = hh[:, H:]

    return pl.pallas_call(
        body,
        grid=(N // R,),
        in_specs=[
            pl.BlockSpec((R, H), lambda i: (i, 0)),
            pl.BlockSpec((H, H), lambda i: (0, 0)),
            pl.BlockSpec((1, H), lambda i: (0, 0)),
            pl.BlockSpec((H, TH), lambda i: (0, 0)),
            pl.BlockSpec((1, TH), lambda i: (0, 0)),
        ],
        out_specs=[
            pl.BlockSpec((R, H), lambda i: (i, 0)),
            pl.BlockSpec((R, TH - H), lambda i: (i, 0)),
        ],
        out_shape=[
            jax.ShapeDtypeStruct((N, H), jnp.float32),
            jax.ShapeDtypeStruct((N, TH - H), jnp.float32),
        ],
    )(s, W1t, b1, W2t, b2)


def _filter_pallas(edge_dist, Wft, bf):
    E, G = edge_dist.shape
    TH = Wft.shape[1]
    H = TH // 3
    R = 2000
    assert E % R == 0

    def body(d_ref, wf_ref, bf_ref, w0_ref, w12_ref):
        ww = jnp.dot(d_ref[...], wf_ref[...], preferred_element_type=jnp.float32)
        ww = ww + bf_ref[...]
        w0_ref[...] = ww[:, :H]
        w12_ref[...] = ww[:, H:]

    return pl.pallas_call(
        body,
        grid=(E // R,),
        in_specs=[
            pl.BlockSpec((R, G), lambda i: (i, 0)),
            pl.BlockSpec((G, TH), lambda i: (0, 0)),
            pl.BlockSpec((1, TH), lambda i: (0, 0)),
        ],
        out_specs=[
            pl.BlockSpec((R, H), lambda i: (i, 0)),
            pl.BlockSpec((R, TH - H), lambda i: (i, 0)),
        ],
        out_shape=[
            jax.ShapeDtypeStruct((E, H), jnp.float32),
            jax.ShapeDtypeStruct((E, TH - H), jnp.float32),
        ],
    )(edge_dist, Wft, bf)


@functools.cache
def _make_sc_kernel(N, E, H):
    assert N % ZR == 0
    assert E % (NTILES * NB) == 0
    nch = N // ZR
    nch_pt = -(-nch // NTILES)
    ept = E // NTILES
    nbatch = ept // NB
    nslc = H // L
    mesh = plsc.VectorSubcoreMesh(core_axis_name="c", subcore_axis_name="s",
                                  num_cores=2, num_subcores=NTILES)

    ring_types = [
        pltpu.VMEM((NB,), jnp.int32),
        pltpu.VMEM((NB,), jnp.int32),
        pltpu.VMEM((NB,), jnp.float32),
        pltpu.VMEM((NB, 2 * H), jnp.float32),
        pltpu.VMEM((NB, 2 * H), jnp.float32),
        pltpu.VMEM((NB, H), jnp.float32),
        pltpu.SemaphoreType.DMA,
        pltpu.SemaphoreType.DMA,
    ]

    @functools.partial(
        pl.kernel,
        out_type=[jax.ShapeDtypeStruct((N, H), jnp.float32)] * 4,
        mesh=mesh,
        scratch_types=[
            pltpu.VMEM_SHARED((N, H), jnp.float32),
            pltpu.VMEM((ZR, H), jnp.float32),
        ] + ring_types * NR,
    )
    def sc_kernel(h0, h12, w0, w12, v0, v1, v2, ev0, ev1, ev2, src, dst,
                  s_out, u0_out, u1_out, u2_out,
                  acc, zbuf, *ringargs):
        sets = [ringargs[i * 8:(i + 1) * 8] for i in range(NR)]
        cid = lax.axis_index("c")
        sid = lax.axis_index("s")
        zero16 = jnp.zeros((L,), jnp.float32)

        def fill_zbuf(i, c):
            for j in range(nslc):
                zbuf[i, pl.ds(L * j, L)] = zero16
            return c

        def zero_acc():
            lax.fori_loop(0, ZR, fill_zbuf, 0)

            def z(i, c):
                cidx = sid + NTILES * i

                @pl.when(cidx < nch)
                def _():
                    pltpu.sync_copy(zbuf, acc.at[pl.ds(cidx * ZR, ZR)])
                return c
            lax.fori_loop(0, nch_pt, z, 0)

        def writeout(out_hbm):
            def wlp(i, c):
                cidx = sid + NTILES * i

                @pl.when(cidx < nch)
                def _():
                    r0 = cidx * ZR
                    pltpu.sync_copy(acc.at[pl.ds(r0, ZR)], zbuf)
                    pltpu.sync_copy(zbuf, out_hbm.at[pl.ds(r0, ZR)])
                return c
            lax.fori_loop(0, nch_pt, wlp, 0)

        def pipelined(nb_total, issue_small, wait_small, issue_big, wait_big,
                      do_batch):
            for j in range(NR):
                if j < nb_total:
                    issue_small(j, j)
            for j in range(NR - 1):
                if j < nb_total:
                    wait_small(j, j)
                    issue_big(j, j)

            def phase(b, slot):
                bn = b + NR - 1
                nslot = (slot + NR - 1) % NR

                @pl.when(bn < nb_total)
                def _():
                    wait_small(bn, nslot)
                    issue_big(bn, nslot)
                wait_big(b, slot)
                do_batch(b, slot)

                @pl.when(b + NR < nb_total)
                def _():
                    issue_small(b + NR, slot)

            def grp(i, c):
                b0 = NR * i
                phase(b0, 0)
                for p in range(1, NR):
                    @pl.when(b0 + p < nb_total)
                    def _(p=p):
                        phase(b0 + p, p)
                return c

            lax.fori_loop(0, -(-nb_total // NR), grp, 0)

        def accum_s():
            def issue_small(b, k):
                srcb, dstb, semS = sets[k][0], sets[k][1], sets[k][6]
                base = sid * ept + b * NB
                pltpu.async_copy(dst.at[pl.ds(base, NB)], dstb, semS)
                pltpu.async_copy(src.at[pl.ds(base, NB)], srcb, semS)

            def wait_small(b, k):
                srcb, dstb, semS = sets[k][0], sets[k][1], sets[k][6]
                base = sid * ept + b * NB
                pltpu.make_async_copy(dst.at[pl.ds(base, NB)], dstb, semS).wait()
                pltpu.make_async_copy(src.at[pl.ds(base, NB)], srcb, semS).wait()

            def issue_big(b, k):
                srcb, semB = sets[k][0], sets[k][7]
                vkbuf, wk = sets[k][5], sets[k + NR // 2][5]
                base = sid * ept + b * NB
                pltpu.async_copy(h0.at[srcb], vkbuf, semB)
                pltpu.async_copy(w0.at[pl.ds(base, NB)], wk, semB)

            def wait_big(b, k):
                srcb, semB = sets[k][0], sets[k][7]
                vkbuf, wk = sets[k][5], sets[k + NR // 2][5]
                base = sid * ept + b * NB
                pltpu.make_async_copy(h0.at[srcb], vkbuf, semB).wait()
                pltpu.make_async_copy(w0.at[pl.ds(base, NB)], wk, semB).wait()

            def do_batch(b, k):
                dstb = sets[k][1]
                vkbuf, wk = sets[k][5], sets[k + NR // 2][5]

                def edge(e, c2):
                    for j in range(nslc):
                        sl = pl.ds(L * j, L)
                        vkbuf[e, sl] = vkbuf[e, sl] * wk[e, sl]
                    return c2

                lax.fori_loop(0, NB, edge, 0)
                pltpu.sync_copy(vkbuf, acc.at[dstb], add=True)

            def pipelined_s():
                nrs = NR // 2
                for j in range(nrs):
                    issue_small(j, j)
                for j in range(nrs - 1):
                    wait_small(j, j)
                    issue_big(j, j)

                def phase(b, slot):
                    bn = b + nrs - 1
                    nslot = (slot + nrs - 1) % nrs

                    @pl.when(bn < nbatch)
                    def _():
                        wait_small(bn, nslot)
                        issue_big(bn, nslot)
                    wait_big(b, slot)
                    do_batch(b, slot)

                    @pl.when(b + nrs < nbatch)
                    def _():
                        issue_small(b + nrs, slot)

                def grp(i, c):
                    b0 = nrs * i
                    phase(b0, 0)
                    for p in range(1, nrs):
                        @pl.when(b0 + p < nbatch)
                        def _(p=p):
                            phase(b0 + p, p)
                    return c

                lax.fori_loop(0, -(-nbatch // nrs), grp, 0)

            pipelined_s()

        def accum_v(vk, evk):
            def issue_small(b, k):
                srcb, dstb, evb = sets[k][0], sets[k][1], sets[k][2]
                semS = sets[k][6]
                base = sid * ept + b * NB
                pltpu.async_copy(dst.at[pl.ds(base, NB)], dstb, semS)
                pltpu.async_copy(src.at[pl.ds(base, NB)], srcb, semS)
                pltpu.async_copy(evk.at[pl.ds(base, NB)], evb, semS)

            def wait_small(b, k):
                srcb, dstb, evb = sets[k][0], sets[k][1], sets[k][2]
                semS = sets[k][6]
                base = sid * ept + b * NB
                pltpu.make_async_copy(dst.at[pl.ds(base, NB)], dstb, semS).wait()
                pltpu.make_async_copy(src.at[pl.ds(base, NB)], srcb, semS).wait()
                pltpu.make_async_copy(evk.at[pl.ds(base, NB)], evb, semS).wait()

            def issue_big(b, k):
                srcb, hbuf, wbuf, vkbuf = (sets[k][0], sets[k][3], sets[k][4],
                                           sets[k][5])
                semB = sets[k][7]
                base = sid * ept + b * NB
                pltpu.async_copy(h12.at[srcb], hbuf, semB)
                pltpu.async_copy(vk.at[srcb], vkbuf, semB)
                pltpu.async_copy(w12.at[pl.ds(base, NB)], wbuf, semB)

            def wait_big(b, k):
                srcb, hbuf, wbuf, vkbuf = (sets[k][0], sets[k][3], sets[k][4],
                                           sets[k][5])
                semB = sets[k][7]
                base = sid * ept + b * NB
                pltpu.make_async_copy(h12.at[srcb], hbuf, semB).wait()
                pltpu.make_async_copy(vk.at[srcb], vkbuf, semB).wait()
                pltpu.make_async_copy(w12.at[pl.ds(base, NB)], wbuf, semB).wait()

            def do_batch(b, k):
                dstb, evb, hbuf, wbuf, vkbuf = (sets[k][1], sets[k][2],
                                                sets[k][3], sets[k][4],
                                                sets[k][5])

                def edge_group(g, c2):
                    ev16 = evb[pl.ds(L * g, L)]
                    evs = [ev16[t] for t in range(L)]

                    def jbody(j, c3):
                        sl = pl.ds(L * j, L)
                        sl2 = pl.ds(H + L * j, L)
                        for t in range(L):
                            e = L * g + t
                            g1 = hbuf[e, sl] * wbuf[e, sl]
                            g2 = hbuf[e, sl2] * wbuf[e, sl2]
                            vkbuf[e, sl] = vkbuf[e, sl] * g1 + evs[t] * g2
                        return c3

                    lax.fori_loop(0, nslc, jbody, 0)
                    return c2

                lax.fori_loop(0, NB // L, edge_group, 0)
                pltpu.sync_copy(vkbuf, acc.at[dstb], add=True)

            pipelined(nbatch, issue_small, wait_small, issue_big, wait_big,
                      do_batch)

        def chunk(accum_fn, out_hbm):
            zero_acc()
            plsc.subcore_barrier()
            accum_fn()
            plsc.subcore_barrier()
            writeout(out_hbm)
            plsc.subcore_barrier()

        @pl.when(cid == 0)
        def _():
            chunk(accum_s, s_out)
            chunk(lambda: accum_v(v0, ev0), u0_out)

        @pl.when(cid == 1)
        def _():
            chunk(lambda: accum_v(v1, ev1), u1_out)
            chunk(lambda: accum_v(v2, ev2), u2_out)

    return sc_kernel


def kernel(s, v, edge_index, edge_dist, edge_vector, W1, b1, W2, b2, Wf, bf):
    N, H = s.shape
    E = edge_index.shape[1]
    src = edge_index[0]
    dst = edge_index[1]

    h0, h12 = _mlp_pallas(s, W1.T, b1.reshape(1, H), W2.T, b2.reshape(1, 3 * H))
    w0, w12 = _filter_pallas(edge_dist, Wf.T, bf.reshape(1, 3 * H))

    v0 = v[:, 0, :]
    v1 = v[:, 1, :]
    v2 = v[:, 2, :]
    ev0 = edge_vector[:, 0]
    ev1 = edge_vector[:, 1]
    ev2 = edge_vector[:, 2]

    s_out, u0, u1, u2 = _make_sc_kernel(N, E, H)(
        h0, h12, w0, w12, v0, v1, v2, ev0, ev1, ev2, src, dst)
    v_out = jnp.stack([u0, u1, u2], axis=1)
    return (s_out, v_out)
